# baseline (speedup 1.0000x reference)
import functools

import jax
import jax.numpy as jnp
from jax import lax
from jax.experimental import pallas as pl
from jax.experimental.pallas import tpu as pltpu
from jax.experimental.pallas import tpu_sc as plsc

NC = 2
NS = 16
NW = NC * NS
LANES = 16

CH = 1024
ZROWS = 800


def _sc_mesh():
  return plsc.VectorSubcoreMesh(
      core_axis_name="c", subcore_axis_name="s", num_cores=NC,
      num_subcores=NS)


_SC_PARAMS = pltpu.CompilerParams(
    use_tc_tiling_on_sc=False, needs_layout_passes=False)


def _scatter_phase(table, tript, npad, tpad, nedge, mp, mx, offi,
                   with_deg, tript_out):
  half = NW // 2
  slab = tpad // half
  nch = slab // CH
  rpt = npad // NS

  out_type = [jax.ShapeDtypeStruct((NC, npad, 16), jnp.float32)]
  if with_deg:
    out_type.append(jax.ShapeDtypeStruct((NC * npad,), jnp.float32))
  if tript_out:
    out_type.append(jax.ShapeDtypeStruct((3 * tpad,), jnp.int32))

  scratch = [
      pltpu.VMEM_SHARED((npad, 16), jnp.float32),
      pltpu.VMEM((ZROWS, 16), jnp.float32),
  ]
  for _ in range(2):
    scratch += [
        pltpu.VMEM((CH,), jnp.int32),
        pltpu.VMEM((CH,), jnp.int32),
        pltpu.VMEM((CH,), jnp.int32),
        pltpu.VMEM((CH,), jnp.int32),
        pltpu.VMEM((CH,), jnp.int32),
        pltpu.VMEM((CH, 16), jnp.float32),
    ]
  scratch += [
      pltpu.SemaphoreType.DMA,
      pltpu.SemaphoreType.DMA,
  ]
  if tript_out:
    scratch.append(pltpu.SemaphoreType.DMA)
  if with_deg:
    scratch += [
        pltpu.VMEM_SHARED((npad,), jnp.float32),
        pltpu.VMEM((rpt,), jnp.float32),
        pltpu.VMEM((128,), jnp.float32),
    ]

  @functools.partial(
      pl.kernel, out_type=tuple(out_type), mesh=_sc_mesh(),
      scratch_types=tuple(scratch), compiler_params=_SC_PARAMS)
  def k(table_h, tript_h, *rest):
    if with_deg and tript_out:
      (hp_h, dp_h, to_h, hacc, z, sb0, pb0, ob0, ib0, db0, rows0,
       sb1, pb1, ob1, ib1, db1, rows1, sem, sem2, sem3, dacc, zd,
       ones) = rest
    else:
      (hp_h, hacc, z, sb0, pb0, ob0, ib0, db0, rows0,
       sb1, pb1, ob1, ib1, db1, rows1, sem, sem2) = rest
      to_h = sem3 = None
    cid = lax.axis_index("c")
    sid = lax.axis_index("s")
    wid = cid * NS + sid
    fwd = wid < half
    q = jnp.where(fwd, wid, wid - half)
    base = q * slab
    offv = jnp.where(fwd, 0, offi)
    iota16 = lax.iota(jnp.int32, 16)

    zero16 = jnp.zeros((16,), jnp.float32)

    def zbody(i, carry):
      z[i] = zero16
      return carry
    lax.fori_loop(0, ZROWS, zbody, 0)
    if with_deg:
      def zdbody(i, carry):
        zd[pl.ds(i * 16, 16)] = zero16
        return carry
      lax.fori_loop(0, rpt // 16, zdbody, 0)
      one16 = jnp.ones((16,), jnp.float32)
      for i in range(8):
        ones[pl.ds(i * 16, 16)] = one16

    for j in range(rpt // ZROWS):
      pltpu.sync_copy(z, hacc.at[pl.ds(sid * rpt + j * ZROWS, ZROWS)])
    if with_deg:
      pltpu.sync_copy(zd, dacc.at[pl.ds(sid * rpt, rpt)])
    plsc.subcore_barrier()

    bufs = ((sb0, pb0, ob0, ib0, db0, rows0),
            (sb1, pb1, ob1, ib1, db1, rows1))

    def load_chunk(c, sb, pb, ob):
      r0 = base + c * CH
      pltpu.sync_copy(tript_h.at[pl.ds(r0, CH)], sb)
      pltpu.sync_copy(tript_h.at[pl.ds(tpad + r0, CH)], pb)
      pltpu.sync_copy(tript_h.at[pl.ds(2 * tpad + r0, CH)], ob)

    def fire_twrites(c, sb, pb, ob):
      r0 = base + c * CH
      pltpu.async_copy(sb, to_h.at[pl.ds(r0, CH)], sem3)
      pltpu.async_copy(pb, to_h.at[pl.ds(tpad + r0, CH)], sem3)
      pltpu.async_copy(ob, to_h.at[pl.ds(2 * tpad + r0, CH)], sem3)

    def wait_twrites(c, sb, pb, ob):
      r0 = base + c * CH
      pltpu.make_async_copy(sb, to_h.at[pl.ds(r0, CH)], sem3).wait()
      pltpu.make_async_copy(pb, to_h.at[pl.ds(tpad + r0, CH)],
                            sem3).wait()
      pltpu.make_async_copy(ob, to_h.at[pl.ds(2 * tpad + r0, CH)],
                            sem3).wait()

    def compute_idx(c, sb, pb, ob, ib, db):
      r0 = base + c * CH

      def cbody(m, carry):
        sl = pl.ds(m * 16, 16)
        sv = sb[sl]
        pv = pb[sl]
        ov = ob[sl]
        xv = jnp.where(fwd, sv, ov)
        dv = jnp.where(fwd, ov, sv)
        iv = pv * mp + xv * mx + offv
        row = r0 + m * 16 + iota16
        valid = row < nedge
        ib[sl] = jnp.where(valid, iv, 0)
        db[sl] = jnp.where(valid, dv, npad - 1)
        return carry
      lax.fori_loop(0, CH // 16, cbody, 0)

    def fire_gathers(ib, rb):
      for j in range(8):
        pltpu.async_copy(
            table_h.at[ib.at[pl.ds(j * 128, 128)]],
            rb.at[pl.ds(j * 128, 128)], sem)

    def wait_gathers(ib, rb):
      for j in range(8):
        pltpu.make_async_copy(
            table_h.at[ib.at[pl.ds(j * 128, 128)]],
            rb.at[pl.ds(j * 128, 128)], sem).wait()

    def fire_scatters(db, rb):
      for j in range(8):
        pltpu.async_copy(rb.at[pl.ds(j * 128, 128)],
                         hacc.at[db.at[pl.ds(j * 128, 128)]], sem2,
                         add=True)
        if with_deg:
          pltpu.async_copy(ones, dacc.at[db.at[pl.ds(j * 128, 128)]],
                           sem2, add=True)

    def wait_scatters(db, rb):
      for j in range(8):
        pltpu.make_async_copy(
            rb.at[pl.ds(j * 128, 128)],
            hacc.at[db.at[pl.ds(j * 128, 128)]], sem2).wait()
        if with_deg:
          pltpu.make_async_copy(
              ones, dacc.at[db.at[pl.ds(j * 128, 128)]], sem2).wait()

    def stage(c, bufset):
      sb, pb, ob, ib, db, rb = bufset
      load_chunk(c, sb, pb, ob)
      compute_idx(c, sb, pb, ob, ib, db)
      fire_gathers(ib, rb)
      if tript_out:
        @pl.when(fwd)
        def _():
          fire_twrites(c, sb, pb, ob)

    stage(0, bufs[0])

    def pair(i, carry):
      for b in range(2):
        c = 2 * i + b
        _, _, _, ib, db, rb = bufs[b]
        sb2, pb2, ob2, ib2, db2, rb2 = bufs[1 - b]
        wait_gathers(ib, rb)

        @pl.when(c + 1 < nch)
        def _():
          @pl.when(c >= 1)
          def _():
            wait_scatters(db2, rb2)
            if tript_out:
              @pl.when(fwd)
              def _():
                wait_twrites(c - 1, sb2, pb2, ob2)
          stage(c + 1, bufs[1 - b])

        fire_scatters(db, rb)
      return carry
    lax.fori_loop(0, nch // 2, pair, 0)
    wait_scatters(db0, rows0)
    wait_scatters(db1, rows1)
    if tript_out:
      @pl.when(fwd)
      def _():
        wait_twrites(nch - 2, sb0, pb0, ob0)
        wait_twrites(nch - 1, sb1, pb1, ob1)

    plsc.subcore_barrier()
    pltpu.sync_copy(hacc.at[pl.ds(sid * rpt, rpt)],
                    hp_h.at[cid, pl.ds(sid * rpt, rpt)])
    if with_deg:
      pltpu.sync_copy(dacc.at[pl.ds(sid * rpt, rpt)],
                      dp_h.at[pl.ds(cid * npad + sid * rpt, rpt)])

  return k(table, tript)


def _dense_phase(hp0, hp1, dpt, w0self, bias0, w1cat, npad, rtot):
  blk = ZROWS
  grid = npad // blk
  wcols = rtot * 16

  def body(hp0_r, hp1_r, dp_r, w0s_r, b0_r, w1_r, nw_r):
    deg = dp_r[:, 0:1] + dp_r[:, 1:2] + 1.0
    invd = 1.0 / deg
    h = jnp.maximum(
        (hp0_r[...] + hp1_r[...] + w0s_r[...]) * invd + b0_r[...], 0.0)
    nw_r[...] = jnp.dot(h, w1_r[...], preferred_element_type=jnp.float32)

  return pl.pallas_call(
      body,
      grid=(grid,),
      in_specs=[
          pl.BlockSpec((blk, 16), lambda i: (i, 0)),
          pl.BlockSpec((blk, 16), lambda i: (i, 0)),
          pl.BlockSpec((blk, 2), lambda i: (i, 0)),
          pl.BlockSpec((blk, 16), lambda i: (i, 0)),
          pl.BlockSpec((1, 16), lambda i: (0, 0)),
          pl.BlockSpec((16, wcols), lambda i: (0, 0)),
      ],
      out_specs=pl.BlockSpec((blk, wcols), lambda i: (i, 0)),
      out_shape=jax.ShapeDtypeStruct((npad, wcols), jnp.float32),
  )(hp0, hp1, dpt, w0self, bias0, w1cat)


def _normalize_phase(cp, nwflat, dp, bias1, npad, rtot):
  rpt = npad // NW
  chunk = min(rpt, ZROWS)
  assert rpt % chunk == 0 and chunk % 160 == 0
  nchunk = rpt // chunk

  @functools.partial(
      pl.kernel,
      out_type=jax.ShapeDtypeStruct((npad, 16), jnp.float32),
      mesh=_sc_mesh(),
      scratch_types=(
          pltpu.VMEM((chunk, 16), jnp.float32),
          pltpu.VMEM((chunk, 16), jnp.float32),
          pltpu.VMEM((chunk, 16), jnp.float32),
          pltpu.VMEM((chunk, 16), jnp.float32),
          pltpu.VMEM((chunk,), jnp.float32),
          pltpu.VMEM((chunk,), jnp.float32),
          pltpu.VMEM((chunk,), jnp.int32),
          pltpu.VMEM((16,), jnp.float32),
          pltpu.SemaphoreType.DMA,
      ),
      compiler_params=_SC_PARAMS)
  def k(cp_h, nwf_h, dp_h, b1_h, h2_h, c0b, c1b, nwb, outb, d0b,
        d1b, nwi, b1v, sem):
    cid = lax.axis_index("c")
    sid = lax.axis_index("s")
    wid = sid * NC + cid
    base = wid * rpt
    iota16 = lax.iota(jnp.int32, 16)
    pltpu.sync_copy(b1_h, b1v)
    b1 = b1v[...]
    for j in range(nchunk):
      r0 = base + j * chunk
      pltpu.sync_copy(cp_h.at[0, pl.ds(r0, chunk)], c0b)
      pltpu.sync_copy(cp_h.at[1, pl.ds(r0, chunk)], c1b)
      pltpu.sync_copy(dp_h.at[pl.ds(r0, chunk)], d0b)
      pltpu.sync_copy(dp_h.at[pl.ds(npad + r0, chunk)], d1b)

      def ibody(m, carry):
        nwi[pl.ds(m * 16, 16)] = (r0 + m * 16 + iota16) * rtot + (
            rtot - 1)
        return carry
      lax.fori_loop(0, chunk // 16, ibody, 0)
      for d in range(chunk // 160):
        pltpu.async_copy(nwf_h.at[nwi.at[pl.ds(d * 160, 160)]],
                         nwb.at[pl.ds(d * 160, 160)], sem)
      for d in range(chunk // 160):
        pltpu.make_async_copy(nwf_h.at[nwi.at[pl.ds(d * 160, 160)]],
                              nwb.at[pl.ds(d * 160, 160)], sem).wait()

      def body(m, carry):
        sl = pl.ds(m * 16, 16)
        dv = 1.0 / (d0b[sl] + d1b[sl] + 1.0)
        for kk in range(16):
          rr = m * 16 + kk
          outb[rr] = (c0b[rr] + c1b[rr] + nwb[rr]) * dv[kk] + b1
        return carry
      lax.fori_loop(0, chunk // 16, body, 0)
      pltpu.sync_copy(outb, h2_h.at[pl.ds(r0, chunk)])

  return k(cp, nwflat, dp, bias1)


def _decoder_phase(h2, batcht, relations, nb):
  qpt = nb // NW

  @functools.partial(
      pl.kernel,
      out_type=jax.ShapeDtypeStruct((nb,), jnp.float32),
      mesh=_sc_mesh(),
      scratch_types=(
          pltpu.VMEM((qpt,), jnp.int32),
          pltpu.VMEM((qpt,), jnp.int32),
          pltpu.VMEM((qpt,), jnp.int32),
          pltpu.VMEM((qpt, 16), jnp.float32),
          pltpu.VMEM((qpt, 16), jnp.float32),
          pltpu.VMEM((qpt, 16), jnp.float32),
          pltpu.VMEM((qpt,), jnp.float32),
          pltpu.SemaphoreType.DMA,
      ),
      compiler_params=_SC_PARAMS)
  def k(h2_h, bt_h, rel_h, sc_h, qsb, qpb, qob, hs, hr, ho, scb, sem):
    cid = lax.axis_index("c")
    sid = lax.axis_index("s")
    wid = sid * NC + cid
    q0 = wid * qpt
    pltpu.sync_copy(bt_h.at[0, pl.ds(q0, qpt)], qsb)
    pltpu.sync_copy(bt_h.at[1, pl.ds(q0, qpt)], qpb)
    pltpu.sync_copy(bt_h.at[2, pl.ds(q0, qpt)], qob)
    descs = []
    for j in range(qpt // 128):
      sl = pl.ds(j * 128, 128)
      descs.append(pltpu.async_copy(
          h2_h.at[qsb.at[sl]], hs.at[sl], sem))
      descs.append(pltpu.async_copy(
          rel_h.at[qpb.at[sl]], hr.at[sl], sem))
      descs.append(pltpu.async_copy(
          h2_h.at[qob.at[sl]], ho.at[sl], sem))
    for d in descs:
      d.wait()
    lane = lax.iota(jnp.int32, 16)
    acc = jnp.zeros((16,), jnp.float32)
    for q in range(qpt):
      v = hs[q] * hr[q] * ho[q]
      s = jnp.sum(v)
      acc = jnp.where(lane == (q % 16), s, acc)
      if q % 16 == 15:
        scb[pl.ds((q // 16) * 16, 16)] = acc
    pltpu.sync_copy(scb, sc_h.at[pl.ds(wid * qpt, qpt)])

  return k(h2, batcht, relations)


def kernel(batch, triples, weights0, bias0, weights1, bias1, relations):
  rtot, n, hid = weights0.shape
  r = relations.shape[0]
  t = triples.shape[0]
  nb = batch.shape[0]
  assert hid == 16 and weights1.shape[2] == 16

  npad = -(-n // 2048) * 2048
  if npad == n:
    npad += 2048
  tpad = -(-t // (NW * CH)) * (NW * CH)

  tript = jnp.pad(triples.T, ((0, 0), (0, tpad - t))).reshape(-1)

  w0flat = weights0.reshape(rtot * n, 16)
  hp, dp, sct = _scatter_phase(w0flat, tript, npad, tpad, t,
                               mp=n, mx=1, offi=r * n, with_deg=True,
                               tript_out=True)

  w0self = jnp.pad(weights0[2 * r], ((0, npad - n), (0, 0)))
  w1cat = weights1.transpose(1, 0, 2).reshape(16, rtot * 16)
  dpt = jnp.stack([dp[:npad], dp[npad:]], axis=1)
  nw = _dense_phase(hp[0], hp[1], dpt, w0self, bias0.reshape(1, 16),
                    w1cat, npad, rtot)

  nwflat = nw.reshape(npad * rtot, 16)
  (cp,) = _scatter_phase(nwflat, sct, npad, tpad, t,
                         mp=1, mx=rtot, offi=r, with_deg=False,
                         tript_out=False)

  h2 = _normalize_phase(cp, nwflat, dp, bias1, npad, rtot)

  batcht = batch.T
  scores = _decoder_phase(h2, batcht, relations, nb)
  return scores


# speedup vs baseline: 1.0438x; 1.0438x over previous
import functools

import jax
import jax.numpy as jnp
from jax import lax
from jax.experimental import pallas as pl
from jax.experimental.pallas import tpu as pltpu
from jax.experimental.pallas import tpu_sc as plsc

NC = 2
NS = 16
NW = NC * NS
LANES = 16

CH = 1024
ZROWS = 800


def _sc_mesh():
  return plsc.VectorSubcoreMesh(
      core_axis_name="c", subcore_axis_name="s", num_cores=NC,
      num_subcores=NS)


_SC_PARAMS = pltpu.CompilerParams(
    use_tc_tiling_on_sc=False, needs_layout_passes=False)


def _scatter_phase(table, tript, npad, tpad, nedge, mp, mx, offi,
                   with_deg, tript_out):
  half = NW // 2
  slab = tpad // half
  nch = slab // CH
  rpt = npad // NS

  out_type = [jax.ShapeDtypeStruct((NC, npad, 16), jnp.float32)]
  if with_deg:
    out_type.append(jax.ShapeDtypeStruct((NC * npad,), jnp.float32))
  if tript_out:
    out_type.append(jax.ShapeDtypeStruct((3 * tpad,), jnp.int32))

  scratch = [
      pltpu.VMEM_SHARED((npad, 16), jnp.float32),
      pltpu.VMEM((ZROWS, 16), jnp.float32),
  ]
  for _ in range(2):
    scratch += [
        pltpu.VMEM((CH,), jnp.int32),
        pltpu.VMEM((CH,), jnp.int32),
        pltpu.VMEM((CH,), jnp.int32),
        pltpu.VMEM((CH,), jnp.int32),
        pltpu.VMEM((CH,), jnp.int32),
        pltpu.VMEM((CH, 16), jnp.float32),
    ]
  scratch += [
      pltpu.SemaphoreType.DMA,
      pltpu.SemaphoreType.DMA,
  ]
  if tript_out:
    scratch.append(pltpu.SemaphoreType.DMA)
  if with_deg:
    scratch += [
        pltpu.VMEM_SHARED((npad,), jnp.float32),
        pltpu.VMEM((rpt,), jnp.float32),
        pltpu.VMEM((128,), jnp.float32),
    ]

  @functools.partial(
      pl.kernel, out_type=tuple(out_type), mesh=_sc_mesh(),
      scratch_types=tuple(scratch), compiler_params=_SC_PARAMS)
  def k(table_h, tript_h, *rest):
    if with_deg and tript_out:
      (hp_h, dp_h, to_h, hacc, z, sb0, pb0, ob0, ib0, db0, rows0,
       sb1, pb1, ob1, ib1, db1, rows1, sem, sem2, sem3, dacc, zd,
       ones) = rest
    else:
      (hp_h, hacc, z, sb0, pb0, ob0, ib0, db0, rows0,
       sb1, pb1, ob1, ib1, db1, rows1, sem, sem2) = rest
      to_h = sem3 = None
    cid = lax.axis_index("c")
    sid = lax.axis_index("s")
    wid = sid * NC + cid
    fwd = wid < half
    q = jnp.where(fwd, wid, wid - half)
    base = q * slab
    offv = jnp.where(fwd, 0, offi)
    iota16 = lax.iota(jnp.int32, 16)

    zero16 = jnp.zeros((16,), jnp.float32)

    def zbody(i, carry):
      z[i] = zero16
      return carry
    lax.fori_loop(0, ZROWS, zbody, 0)
    if with_deg:
      def zdbody(i, carry):
        zd[pl.ds(i * 16, 16)] = zero16
        return carry
      lax.fori_loop(0, rpt // 16, zdbody, 0)
      one16 = jnp.ones((16,), jnp.float32)
      for i in range(8):
        ones[pl.ds(i * 16, 16)] = one16

    for j in range(rpt // ZROWS):
      pltpu.sync_copy(z, hacc.at[pl.ds(sid * rpt + j * ZROWS, ZROWS)])
    if with_deg:
      pltpu.sync_copy(zd, dacc.at[pl.ds(sid * rpt, rpt)])
    plsc.subcore_barrier()

    bufs = ((sb0, pb0, ob0, ib0, db0, rows0),
            (sb1, pb1, ob1, ib1, db1, rows1))

    def load_chunk(c, sb, pb, ob):
      r0 = base + c * CH
      pltpu.sync_copy(tript_h.at[pl.ds(r0, CH)], sb)
      pltpu.sync_copy(tript_h.at[pl.ds(tpad + r0, CH)], pb)
      pltpu.sync_copy(tript_h.at[pl.ds(2 * tpad + r0, CH)], ob)

    def fire_twrites(c, sb, pb, ob):
      r0 = base + c * CH
      pltpu.async_copy(sb, to_h.at[pl.ds(r0, CH)], sem3)
      pltpu.async_copy(pb, to_h.at[pl.ds(tpad + r0, CH)], sem3)
      pltpu.async_copy(ob, to_h.at[pl.ds(2 * tpad + r0, CH)], sem3)

    def wait_twrites(c, sb, pb, ob):
      r0 = base + c * CH
      pltpu.make_async_copy(sb, to_h.at[pl.ds(r0, CH)], sem3).wait()
      pltpu.make_async_copy(pb, to_h.at[pl.ds(tpad + r0, CH)],
                            sem3).wait()
      pltpu.make_async_copy(ob, to_h.at[pl.ds(2 * tpad + r0, CH)],
                            sem3).wait()

    def compute_idx(c, sb, pb, ob, ib, db):
      r0 = base + c * CH

      def cbody(m, carry):
        sl = pl.ds(m * 16, 16)
        sv = sb[sl]
        pv = pb[sl]
        ov = ob[sl]
        xv = jnp.where(fwd, sv, ov)
        dv = jnp.where(fwd, ov, sv)
        iv = pv * mp + xv * mx + offv
        row = r0 + m * 16 + iota16
        valid = row < nedge
        ib[sl] = jnp.where(valid, iv, 0)
        db[sl] = jnp.where(valid, dv, npad - 1)
        return carry
      lax.fori_loop(0, CH // 16, cbody, 0)

    def fire_gathers(ib, rb):
      for j in range(8):
        pltpu.async_copy(
            table_h.at[ib.at[pl.ds(j * 128, 128)]],
            rb.at[pl.ds(j * 128, 128)], sem)

    def wait_gathers(ib, rb):
      for j in range(8):
        pltpu.make_async_copy(
            table_h.at[ib.at[pl.ds(j * 128, 128)]],
            rb.at[pl.ds(j * 128, 128)], sem).wait()

    def fire_scatters(db, rb):
      for j in range(8):
        pltpu.async_copy(rb.at[pl.ds(j * 128, 128)],
                         hacc.at[db.at[pl.ds(j * 128, 128)]], sem2,
                         add=True)
        if with_deg:
          pltpu.async_copy(ones, dacc.at[db.at[pl.ds(j * 128, 128)]],
                           sem2, add=True)

    def wait_scatters(db, rb):
      for j in range(8):
        pltpu.make_async_copy(
            rb.at[pl.ds(j * 128, 128)],
            hacc.at[db.at[pl.ds(j * 128, 128)]], sem2).wait()
        if with_deg:
          pltpu.make_async_copy(
              ones, dacc.at[db.at[pl.ds(j * 128, 128)]], sem2).wait()

    def stage(c, bufset):
      sb, pb, ob, ib, db, rb = bufset
      load_chunk(c, sb, pb, ob)
      compute_idx(c, sb, pb, ob, ib, db)
      fire_gathers(ib, rb)
      if tript_out:
        @pl.when(fwd)
        def _():
          fire_twrites(c, sb, pb, ob)

    stage(0, bufs[0])

    def pair(i, carry):
      for b in range(2):
        c = 2 * i + b
        _, _, _, ib, db, rb = bufs[b]
        sb2, pb2, ob2, ib2, db2, rb2 = bufs[1 - b]
        wait_gathers(ib, rb)

        @pl.when(c + 1 < nch)
        def _():
          @pl.when(c >= 1)
          def _():
            wait_scatters(db2, rb2)
            if tript_out:
              @pl.when(fwd)
              def _():
                wait_twrites(c - 1, sb2, pb2, ob2)
          stage(c + 1, bufs[1 - b])

        fire_scatters(db, rb)
      return carry
    lax.fori_loop(0, nch // 2, pair, 0)
    wait_scatters(db0, rows0)
    wait_scatters(db1, rows1)
    if tript_out:
      @pl.when(fwd)
      def _():
        wait_twrites(nch - 2, sb0, pb0, ob0)
        wait_twrites(nch - 1, sb1, pb1, ob1)

    plsc.subcore_barrier()
    pltpu.sync_copy(hacc.at[pl.ds(sid * rpt, rpt)],
                    hp_h.at[cid, pl.ds(sid * rpt, rpt)])
    if with_deg:
      pltpu.sync_copy(dacc.at[pl.ds(sid * rpt, rpt)],
                      dp_h.at[pl.ds(cid * npad + sid * rpt, rpt)])

  return k(table, tript)


def _dense_phase(hp0, hp1, dpt, w0self, bias0, w1cat, w1self, npad,
                 rtot):
  blk = ZROWS
  grid = npad // blk
  wcols = (rtot - 1) * 16

  def body(hp0_r, hp1_r, dp_r, w0s_r, b0_r, w1_r, w1s_r, nw_r, nws_r):
    deg = dp_r[:, 0:1] + dp_r[:, 1:2] + 1.0
    invd = 1.0 / deg
    h = jnp.maximum(
        (hp0_r[...] + hp1_r[...] + w0s_r[...]) * invd + b0_r[...], 0.0)
    nw = jnp.dot(h, w1_r[...], preferred_element_type=jnp.float32)
    nw_r[...] = nw.reshape(2 * blk, 128)
    nws_r[...] = jnp.dot(h, w1s_r[...], preferred_element_type=jnp.float32)

  return pl.pallas_call(
      body,
      grid=(grid,),
      in_specs=[
          pl.BlockSpec((blk, 16), lambda i: (i, 0)),
          pl.BlockSpec((blk, 16), lambda i: (i, 0)),
          pl.BlockSpec((blk, 2), lambda i: (i, 0)),
          pl.BlockSpec((blk, 16), lambda i: (i, 0)),
          pl.BlockSpec((1, 16), lambda i: (0, 0)),
          pl.BlockSpec((16, wcols), lambda i: (0, 0)),
          pl.BlockSpec((16, 16), lambda i: (0, 0)),
      ],
      out_specs=[
          pl.BlockSpec((2 * blk, 128), lambda i: (i, 0)),
          pl.BlockSpec((blk, 16), lambda i: (i, 0)),
      ],
      out_shape=[
          jax.ShapeDtypeStruct((npad * 2, 128), jnp.float32),
          jax.ShapeDtypeStruct((npad, 16), jnp.float32),
      ],
  )(hp0, hp1, dpt, w0self, bias0, w1cat, w1self)


def _normalize_phase(cp, nws, dp, bias1, npad):
  rpt = npad // NW
  chunk = min(rpt, ZROWS)
  assert rpt % chunk == 0
  nchunk = rpt // chunk

  @functools.partial(
      pl.kernel,
      out_type=jax.ShapeDtypeStruct((npad, 16), jnp.float32),
      mesh=_sc_mesh(),
      scratch_types=(
          pltpu.VMEM((chunk, 16), jnp.float32),
          pltpu.VMEM((chunk, 16), jnp.float32),
          pltpu.VMEM((chunk, 16), jnp.float32),
          pltpu.VMEM((chunk, 16), jnp.float32),
          pltpu.VMEM((chunk,), jnp.float32),
          pltpu.VMEM((chunk,), jnp.float32),
          pltpu.VMEM((16,), jnp.float32),
      ),
      compiler_params=_SC_PARAMS)
  def k(cp_h, nws_h, dp_h, b1_h, h2_h, c0b, c1b, nwb, outb, d0b,
        d1b, b1v):
    cid = lax.axis_index("c")
    sid = lax.axis_index("s")
    wid = sid * NC + cid
    base = wid * rpt
    pltpu.sync_copy(b1_h, b1v)
    b1 = b1v[...]
    for j in range(nchunk):
      r0 = base + j * chunk
      pltpu.sync_copy(cp_h.at[0, pl.ds(r0, chunk)], c0b)
      pltpu.sync_copy(cp_h.at[1, pl.ds(r0, chunk)], c1b)
      pltpu.sync_copy(dp_h.at[pl.ds(r0, chunk)], d0b)
      pltpu.sync_copy(dp_h.at[pl.ds(npad + r0, chunk)], d1b)
      pltpu.sync_copy(nws_h.at[pl.ds(r0, chunk)], nwb)

      def body(m, carry):
        sl = pl.ds(m * 16, 16)
        dv = 1.0 / (d0b[sl] + d1b[sl] + 1.0)
        for kk in range(16):
          rr = m * 16 + kk
          outb[rr] = (c0b[rr] + c1b[rr] + nwb[rr]) * dv[kk] + b1
        return carry
      lax.fori_loop(0, chunk // 16, body, 0)
      pltpu.sync_copy(outb, h2_h.at[pl.ds(r0, chunk)])

  return k(cp, nws, dp, bias1)


def _decoder_phase(h2, batcht, relations, nb):
  qpt = nb // NW

  @functools.partial(
      pl.kernel,
      out_type=jax.ShapeDtypeStruct((nb,), jnp.float32),
      mesh=_sc_mesh(),
      scratch_types=(
          pltpu.VMEM((qpt,), jnp.int32),
          pltpu.VMEM((qpt,), jnp.int32),
          pltpu.VMEM((qpt,), jnp.int32),
          pltpu.VMEM((qpt, 16), jnp.float32),
          pltpu.VMEM((qpt, 16), jnp.float32),
          pltpu.VMEM((qpt, 16), jnp.float32),
          pltpu.VMEM((qpt,), jnp.float32),
          pltpu.SemaphoreType.DMA,
      ),
      compiler_params=_SC_PARAMS)
  def k(h2_h, bt_h, rel_h, sc_h, qsb, qpb, qob, hs, hr, ho, scb, sem):
    cid = lax.axis_index("c")
    sid = lax.axis_index("s")
    wid = sid * NC + cid
    q0 = wid * qpt
    pltpu.sync_copy(bt_h.at[0, pl.ds(q0, qpt)], qsb)
    pltpu.sync_copy(bt_h.at[1, pl.ds(q0, qpt)], qpb)
    pltpu.sync_copy(bt_h.at[2, pl.ds(q0, qpt)], qob)
    descs = []
    for j in range(qpt // 128):
      sl = pl.ds(j * 128, 128)
      descs.append(pltpu.async_copy(
          h2_h.at[qsb.at[sl]], hs.at[sl], sem))
      descs.append(pltpu.async_copy(
          rel_h.at[qpb.at[sl]], hr.at[sl], sem))
      descs.append(pltpu.async_copy(
          h2_h.at[qob.at[sl]], ho.at[sl], sem))
    for d in descs:
      d.wait()
    lane = lax.iota(jnp.int32, 16)
    acc = jnp.zeros((16,), jnp.float32)
    for q in range(qpt):
      v = hs[q] * hr[q] * ho[q]
      s = jnp.sum(v)
      acc = jnp.where(lane == (q % 16), s, acc)
      if q % 16 == 15:
        scb[pl.ds((q // 16) * 16, 16)] = acc
    pltpu.sync_copy(scb, sc_h.at[pl.ds(wid * qpt, qpt)])

  return k(h2, batcht, relations)


def kernel(batch, triples, weights0, bias0, weights1, bias1, relations):
  rtot, n, hid = weights0.shape
  r = relations.shape[0]
  t = triples.shape[0]
  nb = batch.shape[0]
  assert hid == 16 and weights1.shape[2] == 16

  npad = -(-n // 2048) * 2048
  if npad == n:
    npad += 2048
  tpad = -(-t // (NW * CH)) * (NW * CH)

  tript = jnp.pad(triples.T, ((0, 0), (0, tpad - t))).reshape(-1)

  w0flat = weights0.reshape(rtot * n, 16)
  hp, dp, sct = _scatter_phase(w0flat, tript, npad, tpad, t,
                               mp=n, mx=1, offi=r * n, with_deg=True,
                               tript_out=True)

  w0self = jnp.pad(weights0[2 * r], ((0, npad - n), (0, 0)))
  w1cat = weights1[:2 * r].transpose(1, 0, 2).reshape(16, 2 * r * 16)
  w1self = weights1[2 * r]
  dpt = jnp.stack([dp[:npad], dp[npad:]], axis=1)
  nw, nws = _dense_phase(hp[0], hp[1], dpt, w0self,
                         bias0.reshape(1, 16), w1cat, w1self, npad,
                         rtot)

  nwflat = nw.reshape(npad * 2 * r, 16)
  (cp,) = _scatter_phase(nwflat, sct, npad, tpad, t,
                         mp=1, mx=2 * r, offi=r, with_deg=False,
                         tript_out=False)

  h2 = _normalize_phase(cp, nws, dp, bias1, npad)

  batcht = batch.T
  scores = _decoder_phase(h2, batcht, relations, nb)
  return scores
